# baseline (speedup 1.0000x reference)
import jax
import jax.numpy as jnp
from jax import lax
from jax.experimental import pallas as pl
from jax.experimental.pallas import tpu as pltpu
from jax.experimental.pallas import tpu_sc as plsc

_N = 10000
_E = 320000
_H = 256
_C = 40
_CP = 128

_NQ = 4
_QW = _H // _NQ

_NC = 2
_NS = 16
_NW = _NC * _NS
_NP = 10240
_SLC = _NP // _NS

_EPT = _E // _NS
_EPTP = 20480
_CH = 128
_NIT = _EPTP // _CH

_DCH = 80
_DPW = _E // _NW
_DNIT = _DPW // _DCH

_mesh = plsc.VectorSubcoreMesh(core_axis_name="c", subcore_axis_name="s")
_sc_params = pltpu.CompilerParams(use_tc_tiling_on_sc=False)


def _sc_deg_body(dst_hbm, w_hbm, degp_hbm, dflat, wflat, dst2d, w2d,
                 buf_v, deg_sh, sem):
    c = lax.axis_index("c")
    s = lax.axis_index("s")
    wid = c * _NS + s

    pltpu.sync_copy(dst_hbm.at[pl.ds(wid * _DPW, _DPW)], dflat)
    pltpu.sync_copy(w_hbm.at[pl.ds(wid * _DPW, _DPW)], wflat)

    def repack(i, carry):
        for j in range(_DCH // 16):
            sl = pl.ds(j * 16, 16)
            dst2d[i, sl] = dflat[pl.ds(i * _DCH + j * 16, 16)]
            w2d[i, sl] = wflat[pl.ds(i * _DCH + j * 16, 16)]
        return carry

    lax.fori_loop(0, _DNIT, repack, 0)

    def zero(i, carry):
        buf_v[pl.ds(i * 16, 16)] = jnp.zeros((16,), jnp.float32)
        return carry

    lax.fori_loop(0, _SLC // 16, zero, 0)
    pltpu.sync_copy(buf_v, deg_sh.at[pl.ds(s * _SLC, _SLC)])
    plsc.subcore_barrier()

    def group(gi, carry):
        for k in range(5):
            i = gi * 5 + k
            pltpu.async_copy(w2d.at[i], deg_sh.at[dst2d.at[i]], sem,
                             add=True)
        for k in range(5):
            i = gi * 5 + k
            pltpu.make_async_copy(w2d.at[i], deg_sh.at[dst2d.at[i]],
                                  sem).wait()
        return carry

    lax.fori_loop(0, _DNIT // 5, group, 0)
    plsc.subcore_barrier()
    pltpu.sync_copy(deg_sh.at[pl.ds(s * _SLC, _SLC)], buf_v)
    pltpu.sync_copy(buf_v, degp_hbm.at[c, s])


_deg_call = pl.kernel(
    _sc_deg_body,
    out_type=jax.ShapeDtypeStruct((_NC, _NS, _SLC), jnp.float32),
    mesh=_mesh,
    scratch_types=[
        pltpu.VMEM((_DPW,), jnp.int32),
        pltpu.VMEM((_DPW,), jnp.float32),
        pltpu.VMEM((_DNIT, _DCH), jnp.int32),
        pltpu.VMEM((_DNIT, _DCH), jnp.float32),
        pltpu.VMEM((_SLC,), jnp.float32),
        pltpu.VMEM_SHARED((_NP,), jnp.float32),
        pltpu.SemaphoreType.DMA,
    ],
    compiler_params=_sc_params,
)

_DB = 160


def _sc_agg_body(g_hbm, src_hbm, dst_hbm, w_hbm, out_hbm,
                 src_all, dst_all, w_all, idx2, didx2, rows2, dump_v,
                 acc_sh, sem0, sem1):
    c = lax.axis_index("c")
    s = lax.axis_index("s")
    gsem = (sem0, sem1)

    ebase = s * _EPT
    pltpu.sync_copy(src_hbm.at[pl.ds(ebase, _EPT)], src_all.at[pl.ds(0, _EPT)])
    pltpu.sync_copy(dst_hbm.at[pl.ds(ebase, _EPT)], dst_all.at[pl.ds(0, _EPT)])
    pltpu.sync_copy(w_hbm.at[pl.ds(ebase, _EPT)], w_all.at[pl.ds(0, _EPT)])

    def zero_tail(i, carry):
        sl = pl.ds(_EPT + i * 16, 16)
        src_all[sl] = jnp.zeros((16,), jnp.int32)
        dst_all[sl] = jnp.zeros((16,), jnp.int32)
        w_all[sl] = jnp.zeros((16,), jnp.float32)
        return carry

    lax.fori_loop(0, (_EPTP - _EPT) // 16, zero_tail, 0)

    def zero_dump(i, carry):
        for j in range(_QW // 16):
            dump_v[i, pl.ds(j * 16, 16)] = jnp.zeros((16,), jnp.float32)
        return carry

    for p in range(2):
        q = c * 2 + p
        bias = q * _N

        lax.fori_loop(0, _DB, zero_dump, 0)
        for k in range(_SLC // _DB):
            pltpu.sync_copy(dump_v,
                            acc_sh.at[pl.ds(s * _SLC + k * _DB, _DB)])
        plsc.subcore_barrier()

        def build_idx(slot, chunk):
            cb = chunk * _CH
            for k in range(_CH // 16):
                sl = pl.ds(k * 16, 16)
                idx2[slot, sl] = src_all[pl.ds(cb + k * 16, 16)] + bias
                didx2[slot, sl] = dst_all[pl.ds(cb + k * 16, 16)]

        def issue_gather(slot):
            pltpu.async_copy(g_hbm.at[idx2.at[slot]], rows2.at[slot],
                             gsem[slot])

        def wait_gather(slot):
            pltpu.make_async_copy(g_hbm.at[idx2.at[slot]],
                                  rows2.at[slot], gsem[slot]).wait()

        def scale(slot, chunk):
            cb = chunk * _CH

            def sc16(k, c2):
                wvec = w_all[pl.ds(cb + k * 16, 16)]
                for l in range(16):
                    wv = wvec[l]
                    e = k * 16 + l
                    for j in range(_QW // 16):
                        sl = pl.ds(j * 16, 16)
                        rows2[slot, e, sl] = rows2[slot, e, sl] * wv
                return c2

            lax.fori_loop(0, _CH // 16, sc16, 0)

        def scatter(slot):
            pltpu.sync_copy(rows2.at[slot], acc_sh.at[didx2.at[slot]],
                            add=True)

        build_idx(0, 0)
        issue_gather(0)

        def pair(ip, carry):
            c0 = ip * 2
            build_idx(1, c0 + 1)
            issue_gather(1)
            wait_gather(0)
            scale(0, c0)
            scatter(0)

            @pl.when(c0 + 2 < _NIT)
            def _():
                build_idx(0, c0 + 2)
                issue_gather(0)

            wait_gather(1)
            scale(1, c0 + 1)
            scatter(1)
            return carry

        lax.fori_loop(0, _NIT // 2, pair, 0)
        plsc.subcore_barrier()
        for k in range(_SLC // _DB):
            pltpu.sync_copy(acc_sh.at[pl.ds(s * _SLC + k * _DB, _DB)], dump_v)
            pltpu.sync_copy(dump_v,
                            out_hbm.at[q, pl.ds(s * _SLC + k * _DB, _DB)])
        plsc.subcore_barrier()


_agg_call = pl.kernel(
    _sc_agg_body,
    out_type=jax.ShapeDtypeStruct((_NQ, _NP, _QW), jnp.float32),
    mesh=_mesh,
    scratch_types=[
        pltpu.VMEM((_EPTP,), jnp.int32),
        pltpu.VMEM((_EPTP,), jnp.int32),
        pltpu.VMEM((_EPTP,), jnp.float32),
        pltpu.VMEM((2, _CH), jnp.int32),
        pltpu.VMEM((2, _CH), jnp.int32),
        pltpu.VMEM((2, _CH, _QW), jnp.float32),
        pltpu.VMEM((_DB, _QW), jnp.float32),
        pltpu.VMEM_SHARED((_NP, _QW), jnp.float32),
        pltpu.SemaphoreType.DMA,
        pltpu.SemaphoreType.DMA,
    ],
    compiler_params=_sc_params,
)


def _dinv_body(degp_ref, out_ref):
    deg = degp_ref[0] + degp_ref[1] + 1.0
    out_ref[...] = lax.rsqrt(jnp.maximum(deg, 1e-12))[:, None]


_dinv_call = pl.pallas_call(
    _dinv_body,
    out_shape=jax.ShapeDtypeStruct((_NP, 1), jnp.float32),
)

_RB = 1000
_GRID = _N // _RB


def _mm_scale_body(parts_ref, w_ref, b_ref, dinv_ref, out_ref):
    p = parts_ref.shape[0]
    h = b_ref[...].astype(jnp.float32)
    for i in range(p):
        h = h + jnp.dot(parts_ref[i], w_ref[i],
                        preferred_element_type=jnp.float32)
    g = dinv_ref[...] * h
    for q in range(_NQ):
        out_ref[q] = g[:, q * _QW:(q + 1) * _QW]


def _make_mm_scale(p, pw):
    return pl.pallas_call(
        _mm_scale_body,
        grid=(_GRID,),
        in_specs=[
            pl.BlockSpec((p, _RB, pw), lambda i: (0, i, 0)),
            pl.BlockSpec((p, pw, _H), lambda i: (0, 0, 0)),
            pl.BlockSpec((1, _H), lambda i: (0, 0)),
            pl.BlockSpec((_RB, 1), lambda i: (i, 0)),
        ],
        out_specs=pl.BlockSpec((_NQ, _RB, _QW), lambda i: (0, i, 0)),
        out_shape=jax.ShapeDtypeStruct((_NQ, _N, _QW), jnp.float32),
    )


_mm_scale_1 = _make_mm_scale(1, 128)
_mm_scale_2 = _make_mm_scale(_NQ, _QW)


def _post_body(acc_ref, g_ref, dinv_ref, t_ref, out_ref):
    dv = dinv_ref[...][None]
    out_ref[...] = jnp.maximum(
        t_ref[0, 0] * dv * (acc_ref[...] + g_ref[...]), 0.0)


_post_call = pl.pallas_call(
    _post_body,
    grid=(_GRID,),
    in_specs=[
        pl.BlockSpec((_NQ, _RB, _QW), lambda i: (0, i, 0)),
        pl.BlockSpec((_NQ, _RB, _QW), lambda i: (0, i, 0)),
        pl.BlockSpec((_RB, 1), lambda i: (i, 0)),
        pl.BlockSpec((1, 1), lambda i: (0, 0)),
    ],
    out_specs=pl.BlockSpec((_NQ, _RB, _QW), lambda i: (0, i, 0)),
    out_shape=jax.ShapeDtypeStruct((_NQ, _N, _QW), jnp.float32),
)


def _final_body(h1_ref, h2_ref, wout_ref, bout_ref, out_ref):
    z = bout_ref[...].astype(jnp.float32)
    for q in range(_NQ):
        z = z + jnp.dot(h1_ref[q], wout_ref[q],
                        preferred_element_type=jnp.float32)
        z = z + jnp.dot(h2_ref[q], wout_ref[_NQ + q],
                        preferred_element_type=jnp.float32)
    m = jnp.max(z, axis=1, keepdims=True)
    ez = jnp.exp(z - m)
    ls = z - m - jnp.log(jnp.sum(ez, axis=1, keepdims=True))
    out_ref[...] = ls[:, :_C]


_final_call = pl.pallas_call(
    _final_body,
    grid=(_GRID,),
    in_specs=[
        pl.BlockSpec((_NQ, _RB, _QW), lambda i: (0, i, 0)),
        pl.BlockSpec((_NQ, _RB, _QW), lambda i: (0, i, 0)),
        pl.BlockSpec((2 * _NQ, _QW, _CP), lambda i: (0, 0, 0)),
        pl.BlockSpec((1, _CP), lambda i: (0, 0)),
    ],
    out_specs=pl.BlockSpec((_RB, _C), lambda i: (i, 0)),
    out_shape=jax.ShapeDtypeStruct((_N, _C), jnp.float32),
)


def kernel(x, edge_index, edge_attr, W1, b1, t1, W2, b2, t2, Wout, bout):
    src = edge_index[0]
    dst = edge_index[1]
    w = edge_attr

    degp = _deg_call(dst, w)
    dinv = _dinv_call(degp.reshape(_NC, _NP))

    g1 = _mm_scale_1(x[None], W1[None], b1[None], dinv)
    acc1 = _agg_call(g1.reshape(_NQ * _N, _QW), src, dst, w)
    h1 = _post_call(acc1, g1, dinv, t1.reshape(1, 1))

    g2 = _mm_scale_2(h1, W2.reshape(_NQ, _QW, _H), b2[None], dinv)
    acc2 = _agg_call(g2.reshape(_NQ * _N, _QW), src, dst, w)
    h2 = _post_call(acc2, g2, dinv, t2.reshape(1, 1))

    wout_p = jnp.concatenate(
        [Wout, jnp.zeros((2 * _H, _CP - _C), Wout.dtype)], axis=1)
    bout_p = jnp.concatenate(
        [bout, jnp.full((_CP - _C,), -1e30, bout.dtype)])
    return _final_call(h1, h2, wout_p.reshape(2 * _NQ, _QW, _CP),
                       bout_p[None])


# speedup vs baseline: 1.3241x; 1.3241x over previous
import jax
import jax.numpy as jnp
from jax import lax
from jax.experimental import pallas as pl
from jax.experimental.pallas import tpu as pltpu
from jax.experimental.pallas import tpu_sc as plsc

_N = 10000
_E = 320000
_H = 256
_C = 40
_CP = 128

_NQ = 4
_QW = _H // _NQ

_NC = 2
_NS = 16
_NW = _NC * _NS
_NP = 10240
_SLC = _NP // _NS

_EPT = _E // _NS
_CH = 80
_NIT = _EPT // _CH

_DCH = 80
_DPW = _E // _NW
_DNIT = _DPW // _DCH

_mesh = plsc.VectorSubcoreMesh(core_axis_name="c", subcore_axis_name="s")
_sc_params = pltpu.CompilerParams(use_tc_tiling_on_sc=False)


def _sc_deg_body(dst_hbm, w_hbm, degp_hbm, dflat, wflat, dst2d, w2d,
                 buf_v, deg_sh, sem):
    c = lax.axis_index("c")
    s = lax.axis_index("s")
    wid = c * _NS + s

    pltpu.sync_copy(dst_hbm.at[pl.ds(wid * _DPW, _DPW)], dflat)
    pltpu.sync_copy(w_hbm.at[pl.ds(wid * _DPW, _DPW)], wflat)

    def repack(i, carry):
        for j in range(_DCH // 16):
            sl = pl.ds(j * 16, 16)
            dst2d[i, sl] = dflat[pl.ds(i * _DCH + j * 16, 16)]
            w2d[i, sl] = wflat[pl.ds(i * _DCH + j * 16, 16)]
        return carry

    lax.fori_loop(0, _DNIT, repack, 0)

    def zero(i, carry):
        buf_v[pl.ds(i * 16, 16)] = jnp.zeros((16,), jnp.float32)
        return carry

    lax.fori_loop(0, _SLC // 16, zero, 0)
    pltpu.sync_copy(buf_v, deg_sh.at[pl.ds(s * _SLC, _SLC)])
    plsc.subcore_barrier()

    def group(gi, carry):
        for k in range(5):
            i = gi * 5 + k
            pltpu.async_copy(w2d.at[i], deg_sh.at[dst2d.at[i]], sem,
                             add=True)
        for k in range(5):
            i = gi * 5 + k
            pltpu.make_async_copy(w2d.at[i], deg_sh.at[dst2d.at[i]],
                                  sem).wait()
        return carry

    lax.fori_loop(0, _DNIT // 5, group, 0)
    plsc.subcore_barrier()
    pltpu.sync_copy(deg_sh.at[pl.ds(s * _SLC, _SLC)], buf_v)
    pltpu.sync_copy(buf_v, degp_hbm.at[c, s])


_deg_call = pl.kernel(
    _sc_deg_body,
    out_type=jax.ShapeDtypeStruct((_NC, _NS, _SLC), jnp.float32),
    mesh=_mesh,
    scratch_types=[
        pltpu.VMEM((_DPW,), jnp.int32),
        pltpu.VMEM((_DPW,), jnp.float32),
        pltpu.VMEM((_DNIT, _DCH), jnp.int32),
        pltpu.VMEM((_DNIT, _DCH), jnp.float32),
        pltpu.VMEM((_SLC,), jnp.float32),
        pltpu.VMEM_SHARED((_NP,), jnp.float32),
        pltpu.SemaphoreType.DMA,
    ],
    compiler_params=_sc_params,
)

_DB = 160


def _sc_agg_body(g_hbm, src_hbm, dst_hbm, w_hbm, out_hbm,
                 src_all, dst_all, w_all, idx2, didx2, rows2, dump_v,
                 acc_sh, sem0, sem1):
    c = lax.axis_index("c")
    s = lax.axis_index("s")
    gsem = (sem0, sem1)

    ebase = s * _EPT
    pltpu.sync_copy(src_hbm.at[pl.ds(ebase, _EPT)], src_all)
    pltpu.sync_copy(dst_hbm.at[pl.ds(ebase, _EPT)], dst_all)
    pltpu.sync_copy(w_hbm.at[pl.ds(ebase, _EPT)], w_all)

    def zero_dump(i, carry):
        for j in range(_QW // 16):
            dump_v[i, pl.ds(j * 16, 16)] = jnp.zeros((16,), jnp.float32)
        return carry

    for p in range(2):
        q = c * 2 + p
        bias = q * _N

        lax.fori_loop(0, _DB, zero_dump, 0)
        for k in range(_SLC // _DB):
            pltpu.sync_copy(dump_v,
                            acc_sh.at[pl.ds(s * _SLC + k * _DB, _DB)])
        plsc.subcore_barrier()

        def build_idx(slot, chunk):
            cb = chunk * _CH
            for k in range(_CH // 16):
                sl = pl.ds(k * 16, 16)
                idx2[slot, sl] = src_all[pl.ds(cb + k * 16, 16)] + bias
                didx2[slot, sl] = dst_all[pl.ds(cb + k * 16, 16)]

        def issue_gather(slot):
            pltpu.async_copy(g_hbm.at[idx2.at[slot]], rows2.at[slot],
                             gsem[slot])

        def wait_gather(slot):
            pltpu.make_async_copy(g_hbm.at[idx2.at[slot]],
                                  rows2.at[slot], gsem[slot]).wait()

        def scale(slot, chunk):
            cb = chunk * _CH

            def sc16(k, c2):
                wvec = w_all[pl.ds(cb + k * 16, 16)]
                for l in range(16):
                    wv = wvec[l]
                    e = k * 16 + l
                    for j in range(_QW // 16):
                        sl = pl.ds(j * 16, 16)
                        rows2[slot, e, sl] = rows2[slot, e, sl] * wv
                return c2

            lax.fori_loop(0, _CH // 16, sc16, 0)

        def scatter(slot):
            pltpu.sync_copy(rows2.at[slot], acc_sh.at[didx2.at[slot]],
                            add=True)

        build_idx(0, 0)
        issue_gather(0)

        def pair(ip, carry):
            c0 = ip * 2
            build_idx(1, c0 + 1)
            issue_gather(1)
            wait_gather(0)
            scale(0, c0)
            scatter(0)

            @pl.when(c0 + 2 < _NIT)
            def _():
                build_idx(0, c0 + 2)
                issue_gather(0)

            wait_gather(1)
            scale(1, c0 + 1)
            scatter(1)
            return carry

        lax.fori_loop(0, _NIT // 2, pair, 0)
        plsc.subcore_barrier()
        for k in range(_SLC // _DB):
            pltpu.sync_copy(acc_sh.at[pl.ds(s * _SLC + k * _DB, _DB)], dump_v)
            pltpu.sync_copy(dump_v,
                            out_hbm.at[q, pl.ds(s * _SLC + k * _DB, _DB)])
        plsc.subcore_barrier()


_agg_call = pl.kernel(
    _sc_agg_body,
    out_type=jax.ShapeDtypeStruct((_NQ, _NP, _QW), jnp.float32),
    mesh=_mesh,
    scratch_types=[
        pltpu.VMEM((_EPT,), jnp.int32),
        pltpu.VMEM((_EPT,), jnp.int32),
        pltpu.VMEM((_EPT,), jnp.float32),
        pltpu.VMEM((2, _CH), jnp.int32),
        pltpu.VMEM((2, _CH), jnp.int32),
        pltpu.VMEM((2, _CH, _QW), jnp.float32),
        pltpu.VMEM((_DB, _QW), jnp.float32),
        pltpu.VMEM_SHARED((_NP, _QW), jnp.float32),
        pltpu.SemaphoreType.DMA,
        pltpu.SemaphoreType.DMA,
    ],
    compiler_params=_sc_params,
)


def _dinv_body(degp_ref, out_ref):
    deg = degp_ref[0] + degp_ref[1] + 1.0
    out_ref[...] = lax.rsqrt(jnp.maximum(deg, 1e-12))[:, None]


_dinv_call = pl.pallas_call(
    _dinv_body,
    out_shape=jax.ShapeDtypeStruct((_NP, 1), jnp.float32),
)

_RB = 1000
_GRID = _N // _RB


def _mm_scale_body(parts_ref, w_ref, b_ref, dinv_ref, out_ref):
    p = parts_ref.shape[0]
    h = b_ref[...].astype(jnp.float32)
    for i in range(p):
        h = h + jnp.dot(parts_ref[i], w_ref[i],
                        preferred_element_type=jnp.float32)
    g = dinv_ref[...] * h
    for q in range(_NQ):
        out_ref[q] = g[:, q * _QW:(q + 1) * _QW]


def _make_mm_scale(p, pw):
    return pl.pallas_call(
        _mm_scale_body,
        grid=(_GRID,),
        in_specs=[
            pl.BlockSpec((p, _RB, pw), lambda i: (0, i, 0)),
            pl.BlockSpec((p, pw, _H), lambda i: (0, 0, 0)),
            pl.BlockSpec((1, _H), lambda i: (0, 0)),
            pl.BlockSpec((_RB, 1), lambda i: (i, 0)),
        ],
        out_specs=pl.BlockSpec((_NQ, _RB, _QW), lambda i: (0, i, 0)),
        out_shape=jax.ShapeDtypeStruct((_NQ, _N, _QW), jnp.float32),
    )


_mm_scale_1 = _make_mm_scale(1, 128)


def _mm2_fused_body(acc_ref, g_ref, dinv_ref, t_ref, w_ref, b_ref,
                    h1_ref, g2_ref):
    dv = dinv_ref[...][None]
    h1 = jnp.maximum(t_ref[0, 0] * dv * (acc_ref[...] + g_ref[...]), 0.0)
    h1_ref[...] = h1
    h = b_ref[...].astype(jnp.float32)
    for q in range(_NQ):
        h = h + jnp.dot(h1[q], w_ref[q], preferred_element_type=jnp.float32)
    g2 = dinv_ref[...] * h
    for q in range(_NQ):
        g2_ref[q] = g2[:, q * _QW:(q + 1) * _QW]


_mm2_fused = pl.pallas_call(
    _mm2_fused_body,
    grid=(_GRID,),
    in_specs=[
        pl.BlockSpec((_NQ, _RB, _QW), lambda i: (0, i, 0)),
        pl.BlockSpec((_NQ, _RB, _QW), lambda i: (0, i, 0)),
        pl.BlockSpec((_RB, 1), lambda i: (i, 0)),
        pl.BlockSpec((1, 1), lambda i: (0, 0)),
        pl.BlockSpec((_NQ, _QW, _H), lambda i: (0, 0, 0)),
        pl.BlockSpec((1, _H), lambda i: (0, 0)),
    ],
    out_specs=[
        pl.BlockSpec((_NQ, _RB, _QW), lambda i: (0, i, 0)),
        pl.BlockSpec((_NQ, _RB, _QW), lambda i: (0, i, 0)),
    ],
    out_shape=[
        jax.ShapeDtypeStruct((_NQ, _N, _QW), jnp.float32),
        jax.ShapeDtypeStruct((_NQ, _N, _QW), jnp.float32),
    ],
)


def _final_body(h1_ref, acc_ref, g_ref, dinv_ref, t_ref, wout_ref, bout_ref,
                out_ref):
    dv = dinv_ref[...][None]
    h2 = jnp.maximum(t_ref[0, 0] * dv * (acc_ref[...] + g_ref[...]), 0.0)
    z = bout_ref[...].astype(jnp.float32)
    for q in range(_NQ):
        z = z + jnp.dot(h1_ref[q], wout_ref[q],
                        preferred_element_type=jnp.float32)
        z = z + jnp.dot(h2[q], wout_ref[_NQ + q],
                        preferred_element_type=jnp.float32)
    m = jnp.max(z, axis=1, keepdims=True)
    ez = jnp.exp(z - m)
    ls = z - m - jnp.log(jnp.sum(ez, axis=1, keepdims=True))
    out_ref[...] = ls[:, :_C]


_final_call = pl.pallas_call(
    _final_body,
    grid=(_GRID,),
    in_specs=[
        pl.BlockSpec((_NQ, _RB, _QW), lambda i: (0, i, 0)),
        pl.BlockSpec((_NQ, _RB, _QW), lambda i: (0, i, 0)),
        pl.BlockSpec((_NQ, _RB, _QW), lambda i: (0, i, 0)),
        pl.BlockSpec((_RB, 1), lambda i: (i, 0)),
        pl.BlockSpec((1, 1), lambda i: (0, 0)),
        pl.BlockSpec((2 * _NQ, _QW, _CP), lambda i: (0, 0, 0)),
        pl.BlockSpec((1, _CP), lambda i: (0, 0)),
    ],
    out_specs=pl.BlockSpec((_RB, _C), lambda i: (i, 0)),
    out_shape=jax.ShapeDtypeStruct((_N, _C), jnp.float32),
)


def kernel(x, edge_index, edge_attr, W1, b1, t1, W2, b2, t2, Wout, bout):
    src = edge_index[0]
    dst = edge_index[1]
    w = edge_attr

    degp = _deg_call(dst, w)
    dinv = _dinv_call(degp.reshape(_NC, _NP))

    g1 = _mm_scale_1(x[None], W1[None], b1[None], dinv)
    acc1 = _agg_call(g1.reshape(_NQ * _N, _QW), src, dst, w)
    h1, g2 = _mm2_fused(acc1, g1, dinv, t1.reshape(1, 1),
                        W2.reshape(_NQ, _QW, _H), b2[None])
    acc2 = _agg_call(g2.reshape(_NQ * _N, _QW), src, dst, w)

    wout_p = jnp.concatenate(
        [Wout, jnp.zeros((2 * _H, _CP - _C), Wout.dtype)], axis=1)
    bout_p = jnp.concatenate(
        [bout, jnp.full((_CP - _C,), -1e30, bout.dtype)])
    return _final_call(h1, acc2, g2, dinv, t2.reshape(1, 1),
                       wout_p.reshape(2 * _NQ, _QW, _CP), bout_p[None])


# speedup vs baseline: 1.3286x; 1.0034x over previous
import jax
import jax.numpy as jnp
from jax import lax
from jax.experimental import pallas as pl
from jax.experimental.pallas import tpu as pltpu
from jax.experimental.pallas import tpu_sc as plsc

_N = 10000
_E = 320000
_H = 256
_C = 40
_CP = 128

_NQ = 4
_QW = _H // _NQ

_NC = 2
_NS = 16
_NW = _NC * _NS
_NP = 10240
_SLC = _NP // _NS

_EPT = _E // _NS
_CH = 80
_NIT = _EPT // _CH

_DCH = 80
_DPW = _E // _NW
_DNIT = _DPW // _DCH

_mesh = plsc.VectorSubcoreMesh(core_axis_name="c", subcore_axis_name="s")
_sc_params = pltpu.CompilerParams(use_tc_tiling_on_sc=False)


def _sc_deg_body(dst_hbm, w_hbm, degp_hbm, dflat, wflat, dst2d, w2d,
                 buf_v, deg_sh, sem):
    c = lax.axis_index("c")
    s = lax.axis_index("s")
    wid = c * _NS + s

    pltpu.sync_copy(dst_hbm.at[pl.ds(wid * _DPW, _DPW)], dflat)
    pltpu.sync_copy(w_hbm.at[pl.ds(wid * _DPW, _DPW)], wflat)

    def repack(i, carry):
        for j in range(_DCH // 16):
            sl = pl.ds(j * 16, 16)
            dst2d[i, sl] = dflat[pl.ds(i * _DCH + j * 16, 16)]
            w2d[i, sl] = wflat[pl.ds(i * _DCH + j * 16, 16)]
        return carry

    lax.fori_loop(0, _DNIT, repack, 0)

    def zero(i, carry):
        buf_v[pl.ds(i * 16, 16)] = jnp.zeros((16,), jnp.float32)
        return carry

    lax.fori_loop(0, _SLC // 16, zero, 0)
    pltpu.sync_copy(buf_v, deg_sh.at[pl.ds(s * _SLC, _SLC)])
    plsc.subcore_barrier()

    def group(gi, carry):
        for k in range(5):
            i = gi * 5 + k
            pltpu.async_copy(w2d.at[i], deg_sh.at[dst2d.at[i]], sem,
                             add=True)
        for k in range(5):
            i = gi * 5 + k
            pltpu.make_async_copy(w2d.at[i], deg_sh.at[dst2d.at[i]],
                                  sem).wait()
        return carry

    lax.fori_loop(0, _DNIT // 5, group, 0)
    plsc.subcore_barrier()
    pltpu.sync_copy(deg_sh.at[pl.ds(s * _SLC, _SLC)], buf_v)
    pltpu.sync_copy(buf_v, degp_hbm.at[c, s])


_deg_call = pl.kernel(
    _sc_deg_body,
    out_type=jax.ShapeDtypeStruct((_NC, _NS, _SLC), jnp.float32),
    mesh=_mesh,
    scratch_types=[
        pltpu.VMEM((_DPW,), jnp.int32),
        pltpu.VMEM((_DPW,), jnp.float32),
        pltpu.VMEM((_DNIT, _DCH), jnp.int32),
        pltpu.VMEM((_DNIT, _DCH), jnp.float32),
        pltpu.VMEM((_SLC,), jnp.float32),
        pltpu.VMEM_SHARED((_NP,), jnp.float32),
        pltpu.SemaphoreType.DMA,
    ],
    compiler_params=_sc_params,
)

_DB = 160


def _sc_agg_body(g_hbm, src_hbm, dst_hbm, w_hbm, out_hbm,
                 src_all, dst_all, w_all, idx2, didx2, rows2, dump_v,
                 acc_sh, sem0, sem1):
    c = lax.axis_index("c")
    s = lax.axis_index("s")
    gsem = (sem0, sem1)

    ebase = s * _EPT
    pltpu.sync_copy(src_hbm.at[pl.ds(ebase, _EPT)], src_all)
    pltpu.sync_copy(dst_hbm.at[pl.ds(ebase, _EPT)], dst_all)
    pltpu.sync_copy(w_hbm.at[pl.ds(ebase, _EPT)], w_all)

    def zero_dump(i, carry):
        for j in range(_QW // 16):
            dump_v[i, pl.ds(j * 16, 16)] = jnp.zeros((16,), jnp.float32)
        return carry

    for p in range(2):
        q = c * 2 + p
        bias = q * _N

        lax.fori_loop(0, _DB, zero_dump, 0)
        for k in range(_SLC // _DB):
            pltpu.sync_copy(dump_v,
                            acc_sh.at[pl.ds(s * _SLC + k * _DB, _DB)])
        plsc.subcore_barrier()

        def build_idx(slot, chunk):
            cb = chunk * _CH
            for k in range(_CH // 16):
                sl = pl.ds(k * 16, 16)
                idx2[slot, sl] = src_all[pl.ds(cb + k * 16, 16)] + bias
                didx2[slot, sl] = dst_all[pl.ds(cb + k * 16, 16)]

        def issue_gather(slot):
            pltpu.async_copy(g_hbm.at[idx2.at[slot]], rows2.at[slot],
                             gsem[slot])

        def wait_gather(slot):
            pltpu.make_async_copy(g_hbm.at[idx2.at[slot]],
                                  rows2.at[slot], gsem[slot]).wait()

        def scale(slot, chunk):
            cb = chunk * _CH

            def sc16(k, c2):
                wvec = w_all[pl.ds(cb + k * 16, 16)]
                for l in range(16):
                    wv = wvec[l]
                    e = k * 16 + l
                    for j in range(_QW // 16):
                        sl = pl.ds(j * 16, 16)
                        rows2[slot, e, sl] = rows2[slot, e, sl] * wv
                return c2

            lax.fori_loop(0, _CH // 16, sc16, 0)

        def scatter(slot):
            pltpu.sync_copy(rows2.at[slot], acc_sh.at[didx2.at[slot]],
                            add=True)

        build_idx(0, 0)
        issue_gather(0)

        def pair(ip, carry):
            c0 = ip * 2
            build_idx(1, c0 + 1)
            issue_gather(1)
            wait_gather(0)
            scale(0, c0)
            scatter(0)

            @pl.when(c0 + 2 < _NIT)
            def _():
                build_idx(0, c0 + 2)
                issue_gather(0)

            wait_gather(1)
            scale(1, c0 + 1)
            scatter(1)
            return carry

        lax.fori_loop(0, _NIT // 2, pair, 0)
        plsc.subcore_barrier()
        for k in range(_SLC // _DB):
            pltpu.sync_copy(acc_sh.at[pl.ds(s * _SLC + k * _DB, _DB)], dump_v)
            pltpu.sync_copy(dump_v,
                            out_hbm.at[q, pl.ds(s * _SLC + k * _DB, _DB)])
        plsc.subcore_barrier()


_agg_call = pl.kernel(
    _sc_agg_body,
    out_type=jax.ShapeDtypeStruct((_NQ, _NP, _QW), jnp.float32),
    mesh=_mesh,
    scratch_types=[
        pltpu.VMEM((_EPT,), jnp.int32),
        pltpu.VMEM((_EPT,), jnp.int32),
        pltpu.VMEM((_EPT,), jnp.float32),
        pltpu.VMEM((2, _CH), jnp.int32),
        pltpu.VMEM((2, _CH), jnp.int32),
        pltpu.VMEM((2, _CH, _QW), jnp.float32),
        pltpu.VMEM((_DB, _QW), jnp.float32),
        pltpu.VMEM_SHARED((_NP, _QW), jnp.float32),
        pltpu.SemaphoreType.DMA,
        pltpu.SemaphoreType.DMA,
    ],
    compiler_params=_sc_params,
)


def _dinv_body(degp_ref, out_ref):
    deg = degp_ref[0] + degp_ref[1] + 1.0
    out_ref[...] = lax.rsqrt(jnp.maximum(deg, 1e-12))[:, None]


_dinv_call = pl.pallas_call(
    _dinv_body,
    out_shape=jax.ShapeDtypeStruct((_NP, 1), jnp.float32),
)

_RB = 2000
_GRID = _N // _RB


def _mm_scale_body(parts_ref, w_ref, b_ref, dinv_ref, out_ref):
    p = parts_ref.shape[0]
    h = b_ref[...].astype(jnp.float32)
    for i in range(p):
        h = h + jnp.dot(parts_ref[i], w_ref[i],
                        preferred_element_type=jnp.float32)
    g = dinv_ref[...] * h
    for q in range(_NQ):
        out_ref[q] = g[:, q * _QW:(q + 1) * _QW]


def _make_mm_scale(p, pw):
    return pl.pallas_call(
        _mm_scale_body,
        grid=(_GRID,),
        in_specs=[
            pl.BlockSpec((p, _RB, pw), lambda i: (0, i, 0)),
            pl.BlockSpec((p, pw, _H), lambda i: (0, 0, 0)),
            pl.BlockSpec((1, _H), lambda i: (0, 0)),
            pl.BlockSpec((_RB, 1), lambda i: (i, 0)),
        ],
        out_specs=pl.BlockSpec((_NQ, _RB, _QW), lambda i: (0, i, 0)),
        out_shape=jax.ShapeDtypeStruct((_NQ, _N, _QW), jnp.float32),
    )


_mm_scale_1 = _make_mm_scale(1, 128)


def _mm2_fused_body(acc_ref, g_ref, dinv_ref, t_ref, w_ref, b_ref,
                    h1_ref, g2_ref):
    dv = dinv_ref[...][None]
    h1 = jnp.maximum(t_ref[0, 0] * dv * (acc_ref[...] + g_ref[...]), 0.0)
    h1_ref[...] = h1
    h = b_ref[...].astype(jnp.float32)
    for q in range(_NQ):
        h = h + jnp.dot(h1[q], w_ref[q], preferred_element_type=jnp.float32)
    g2 = dinv_ref[...] * h
    for q in range(_NQ):
        g2_ref[q] = g2[:, q * _QW:(q + 1) * _QW]


_mm2_fused = pl.pallas_call(
    _mm2_fused_body,
    grid=(_GRID,),
    in_specs=[
        pl.BlockSpec((_NQ, _RB, _QW), lambda i: (0, i, 0)),
        pl.BlockSpec((_NQ, _RB, _QW), lambda i: (0, i, 0)),
        pl.BlockSpec((_RB, 1), lambda i: (i, 0)),
        pl.BlockSpec((1, 1), lambda i: (0, 0)),
        pl.BlockSpec((_NQ, _QW, _H), lambda i: (0, 0, 0)),
        pl.BlockSpec((1, _H), lambda i: (0, 0)),
    ],
    out_specs=[
        pl.BlockSpec((_NQ, _RB, _QW), lambda i: (0, i, 0)),
        pl.BlockSpec((_NQ, _RB, _QW), lambda i: (0, i, 0)),
    ],
    out_shape=[
        jax.ShapeDtypeStruct((_NQ, _N, _QW), jnp.float32),
        jax.ShapeDtypeStruct((_NQ, _N, _QW), jnp.float32),
    ],
)


def _final_body(h1_ref, acc_ref, g_ref, dinv_ref, t_ref, wout_ref, bout_ref,
                out_ref):
    dv = dinv_ref[...][None]
    h2 = jnp.maximum(t_ref[0, 0] * dv * (acc_ref[...] + g_ref[...]), 0.0)
    z = bout_ref[...].astype(jnp.float32)
    for q in range(_NQ):
        z = z + jnp.dot(h1_ref[q], wout_ref[q],
                        preferred_element_type=jnp.float32)
        z = z + jnp.dot(h2[q], wout_ref[_NQ + q],
                        preferred_element_type=jnp.float32)
    m = jnp.max(z, axis=1, keepdims=True)
    ez = jnp.exp(z - m)
    ls = z - m - jnp.log(jnp.sum(ez, axis=1, keepdims=True))
    out_ref[...] = ls[:, :_C]


_final_call = pl.pallas_call(
    _final_body,
    grid=(_GRID,),
    in_specs=[
        pl.BlockSpec((_NQ, _RB, _QW), lambda i: (0, i, 0)),
        pl.BlockSpec((_NQ, _RB, _QW), lambda i: (0, i, 0)),
        pl.BlockSpec((_NQ, _RB, _QW), lambda i: (0, i, 0)),
        pl.BlockSpec((_RB, 1), lambda i: (i, 0)),
        pl.BlockSpec((1, 1), lambda i: (0, 0)),
        pl.BlockSpec((2 * _NQ, _QW, _CP), lambda i: (0, 0, 0)),
        pl.BlockSpec((1, _CP), lambda i: (0, 0)),
    ],
    out_specs=pl.BlockSpec((_RB, _C), lambda i: (i, 0)),
    out_shape=jax.ShapeDtypeStruct((_N, _C), jnp.float32),
)


def kernel(x, edge_index, edge_attr, W1, b1, t1, W2, b2, t2, Wout, bout):
    src = edge_index[0]
    dst = edge_index[1]
    w = edge_attr

    degp = _deg_call(dst, w)
    dinv = _dinv_call(degp.reshape(_NC, _NP))

    g1 = _mm_scale_1(x[None], W1[None], b1[None], dinv)
    acc1 = _agg_call(g1.reshape(_NQ * _N, _QW), src, dst, w)
    h1, g2 = _mm2_fused(acc1, g1, dinv, t1.reshape(1, 1),
                        W2.reshape(_NQ, _QW, _H), b2[None])
    acc2 = _agg_call(g2.reshape(_NQ * _N, _QW), src, dst, w)

    wout_p = jnp.concatenate(
        [Wout, jnp.zeros((2 * _H, _CP - _C), Wout.dtype)], axis=1)
    bout_p = jnp.concatenate(
        [bout, jnp.full((_CP - _C,), -1e30, bout.dtype)])
    return _final_call(h1, acc2, g2, dinv, t2.reshape(1, 1),
                       wout_p.reshape(2 * _NQ, _QW, _CP), bout_p[None])
